# Initial kernel scaffold; baseline (speedup 1.0000x reference)
import jax
import jax.numpy as jnp
from jax.experimental import pallas as pl


def kernel(x, weight, bias):
    raise NotImplementedError("write your pallas kernel here")


# speedup vs baseline: 2.5000x; 2.5000x over previous
import jax
import jax.numpy as jnp
from jax import lax
from jax.experimental import pallas as pl
from jax.experimental.pallas import tpu as pltpu


def _round_up(x, m):
    return ((x + m - 1) // m) * m


def _linear_fused_kernel(x_ref, w_ref, b_ref, o_ref):
    acc = lax.dot_general(
        x_ref[...], w_ref[...],
        dimension_numbers=(((1,), (1,)), ((), ())),
        preferred_element_type=jnp.float32,
    )
    o_ref[...] = (acc + b_ref[...]).astype(o_ref.dtype)


def kernel(x, weight, bias):
    B, in_f = x.shape
    out_f, in_f2 = weight.shape
    assert in_f == in_f2, (x.shape, weight.shape)

    itemsize = jnp.dtype(x.dtype).itemsize

    in_p = _round_up(in_f, 128)
    tm = min(1024, _round_up(B, 8))
    tn = min(512, _round_up(out_f, 128))
    while tm > 8 and 2 * (tm * in_p + tn * in_p + tm * tn) * itemsize > (56 << 20):
        tm //= 2
    tm = max(tm, 8)

    B_p = _round_up(B, tm)
    out_p = _round_up(out_f, tn)

    if (B_p, in_p) != (B, in_f):
        x = jnp.pad(x, ((0, B_p - B), (0, in_p - in_f)))
    if (out_p, in_p) != (out_f, in_f):
        weight = jnp.pad(weight, ((0, out_p - out_f), (0, in_p - in_f)))
    b2d = bias.reshape(1, out_f)
    if out_p != out_f:
        b2d = jnp.pad(b2d, ((0, 0), (0, out_p - out_f)))

    grid_m = B_p // tm
    grid_n = out_p // tn

    cost = pl.CostEstimate(
        flops=2 * B_p * in_p * out_p,
        transcendentals=0,
        bytes_accessed=(B_p * in_p + grid_m * out_p * in_p
                        + out_p + B_p * out_p) * itemsize,
    )

    io_bytes = 2 * (tm * in_p + tn * in_p + tn + tm * tn) * itemsize
    vmem_limit = int(min(io_bytes + (4 << 20), 62 << 20))

    out = pl.pallas_call(
        _linear_fused_kernel,
        out_shape=jax.ShapeDtypeStruct((B_p, out_p), x.dtype),
        grid_spec=pltpu.PrefetchScalarGridSpec(
            num_scalar_prefetch=0,
            grid=(grid_m, grid_n),
            in_specs=[
                pl.BlockSpec((tm, in_p), lambda i, j: (i, 0)),
                pl.BlockSpec((tn, in_p), lambda i, j: (j, 0)),
                pl.BlockSpec((1, tn), lambda i, j: (0, j)),
            ],
            out_specs=pl.BlockSpec((tm, tn), lambda i, j: (i, j)),
        ),
        compiler_params=pltpu.CompilerParams(
            dimension_semantics=("parallel", "arbitrary"),
            vmem_limit_bytes=vmem_limit,
        ),
        cost_estimate=cost,
    )(x, weight, b2d)

    if (B_p, out_p) != (B, out_f):
        out = out[:B, :out_f]
    return out
